# baseline (speedup 1.0000x reference)
import functools

import jax
import jax.numpy as jnp
from jax import lax
from jax.experimental import pallas as pl
from jax.experimental.pallas import tpu as pltpu
from jax.experimental.pallas import tpu_sc as plsc

ALPHA = 0.2


def kernel(x, edge_index, W, Wb, a_w, a_b):
    N, D = x.shape
    H, _, DH = W.shape
    E = edge_index.shape[1]
    F = H * DH
    SW = 16

    Wcat = jnp.transpose(W, (1, 0, 2)).reshape(D, F)
    bcat = Wb.reshape(1, F).astype(jnp.float32)
    eyeH = jnp.eye(H, dtype=jnp.float32)
    A_src = (a_w[:, :DH][:, :, None] * eyeH[:, None, :]).reshape(F, H)
    A_dst = (a_w[:, DH:][:, :, None] * eyeH[:, None, :]).reshape(F, H)
    Acat = jnp.concatenate(
        [A_src, A_dst, jnp.zeros((F, SW - 2 * H), jnp.float32)], axis=1)
    ab16 = jnp.concatenate(
        [jnp.zeros((H,), jnp.float32), a_b,
         jnp.zeros((SW - 2 * H,), jnp.float32)]).reshape(1, SW)

    def proj_body(x_ref, wc_ref, bc_ref, ac_ref, ab_ref, wh_ref, sp_ref):
        xv = x_ref[...]
        wh = lax.dot(xv, wc_ref[...], precision=lax.Precision.HIGHEST,
                     preferred_element_type=jnp.float32) + bc_ref[...]
        wh_ref[...] = wh
        sp_ref[...] = lax.dot(wh, ac_ref[...], precision=lax.Precision.HIGHEST,
                              preferred_element_type=jnp.float32) + ab_ref[...]

    whflat, s16 = pl.pallas_call(
        proj_body,
        out_shape=(jax.ShapeDtypeStruct((N, F), jnp.float32),
                   jax.ShapeDtypeStruct((N, SW), jnp.float32)),
    )(x, Wcat, bcat, Acat, ab16)

    NC, NS = 2, 16
    NW = NC * NS
    e_per_w = E // NW
    CH = 80
    n_chunks = e_per_w // CH
    rows_per_tile = N // NS
    nfull = rows_per_tile // CH
    rem = rows_per_tile - nfull * CH

    src = edge_index[0]
    dst = edge_index[1]

    mesh = plsc.VectorSubcoreMesh(core_axis_name="c", subcore_axis_name="s")

    @functools.partial(
        pl.kernel, mesh=mesh,
        compiler_params=pltpu.CompilerParams(use_tc_tiling_on_sc=False,
                                             needs_layout_passes=False),
        out_type=(jax.ShapeDtypeStruct((NC, N, F), jnp.float32),
                  jax.ShapeDtypeStruct((NC, N, SW), jnp.float32)),
        scratch_types=[
            pltpu.VMEM((2, CH, F), jnp.float32),
            pltpu.VMEM((2, CH, SW), jnp.float32),
            pltpu.VMEM((2, CH, SW), jnp.float32),
            pltpu.VMEM((2, CH, SW), jnp.float32),
            pltpu.VMEM((2, CH), jnp.int32),
            pltpu.VMEM((2, CH), jnp.int32),
            pltpu.VMEM((2, CH), jnp.int32),
            pltpu.VMEM_SHARED((N, F), jnp.float32),
            pltpu.VMEM_SHARED((N, SW), jnp.float32),
            pltpu.SemaphoreType.DMA((2,)),
            pltpu.SemaphoreType.DMA((2,)),
            pltpu.SemaphoreType.DMA((2,)),
            pltpu.SemaphoreType.DMA((2,)),
            pltpu.SemaphoreType.DMA((2,)),
            pltpu.SemaphoreType.DMA((2,)),
            pltpu.SemaphoreType.DMA((2,)),
        ],
    )
    def sc_pass(src_hbm, dst_hbm, s16_hbm, wh_hbm, oacc_hbm, oden_hbm,
                rows_v, exb, ssrc16, sdst16, srcv, dstv, dstv_scat, acc, accden,
                gsem_r, gsem_s, gsem_d, ssem_r, ssem_d, isem_s, isem_d):
        cid = lax.axis_index("c")
        sid = lax.axis_index("s")
        wid = cid * NS + sid
        base = wid * e_per_w
        zero16 = jnp.zeros((16,), jnp.float32)

        @pl.loop(0, CH)
        def _(r):
            for c in range(F // 16):
                rows_v[0, r, pl.ds(c * 16, 16)] = zero16
            exb[0, r, :] = zero16
            exb[1, r, :] = zero16

        r0 = sid * rows_per_tile

        @pl.loop(0, nfull)
        def _(i):
            pltpu.sync_copy(rows_v.at[0], acc.at[pl.ds(r0 + i * CH, CH)])
            pltpu.sync_copy(exb.at[0], accden.at[pl.ds(r0 + i * CH, CH)])

        if rem:
            pltpu.sync_copy(rows_v.at[0, pl.ds(0, rem)],
                            acc.at[pl.ds(r0 + nfull * CH, rem)])
            pltpu.sync_copy(exb.at[0, pl.ds(0, rem)],
                            accden.at[pl.ds(r0 + nfull * CH, rem)])
        plsc.subcore_barrier()

        def issue_idx(b, j):
            eb = base + j * CH
            pltpu.async_copy(src_hbm.at[pl.ds(eb, CH)], srcv.at[b], isem_s.at[b])
            pltpu.async_copy(dst_hbm.at[pl.ds(eb, CH)], dstv.at[b], isem_d.at[b])

        def wait_idx(b):
            pltpu.make_async_copy(src_hbm.at[pl.ds(0, CH)], srcv.at[b],
                                  isem_s.at[b]).wait()
            pltpu.make_async_copy(dst_hbm.at[pl.ds(0, CH)], dstv.at[b],
                                  isem_d.at[b]).wait()

        def issue_gathers(b):
            pltpu.async_copy(wh_hbm.at[srcv.at[b]], rows_v.at[b], gsem_r.at[b])
            pltpu.async_copy(s16_hbm.at[srcv.at[b]], ssrc16.at[b], gsem_s.at[b])
            pltpu.async_copy(s16_hbm.at[dstv.at[b]], sdst16.at[b], gsem_d.at[b])

        def wait_gathers(b):
            pltpu.make_async_copy(wh_hbm.at[srcv.at[b]], rows_v.at[b],
                                  gsem_r.at[b]).wait()
            pltpu.make_async_copy(s16_hbm.at[srcv.at[b]], ssrc16.at[b],
                                  gsem_s.at[b]).wait()
            pltpu.make_async_copy(s16_hbm.at[dstv.at[b]], sdst16.at[b],
                                  gsem_d.at[b]).wait()

        def wait_scatters(b):
            pltpu.make_async_copy(rows_v.at[b], acc.at[dstv_scat.at[b]],
                                  ssem_r.at[b]).wait()
            pltpu.make_async_copy(exb.at[b], accden.at[dstv_scat.at[b]],
                                  ssem_d.at[b]).wait()

        def copy_dstv(b):
            @pl.loop(0, CH // 16)
            def _(g):
                dstv_scat[b, pl.ds(g * 16, 16)] = dstv[b, pl.ds(g * 16, 16)]

        def compute_and_scatter(b):
            @pl.loop(0, CH // 16)
            def _(g):
                iota = lax.iota(jnp.int32, 16)
                e_vec = g * 16 + iota
                av = [plsc.load_gather(ssrc16.at[b],
                                       [e_vec, jnp.full((16,), h, jnp.int32)])
                      for h in range(H)]
                bv = [plsc.load_gather(sdst16.at[b],
                                       [e_vec, jnp.full((16,), H + h, jnp.int32)])
                      for h in range(H)]
                zv = [a + bb for a, bb in zip(av, bv)]
                ev = [jnp.exp(jnp.maximum(z, z * ALPHA)) for z in zv]
                for h in range(H):
                    plsc.store_scatter(exb.at[b],
                                       [e_vec, jnp.full((16,), h, jnp.int32)],
                                       ev[h])

            @pl.loop(0, CH, unroll=4)
            def _(e):
                exvec = exb[b, e, :]
                ss = tuple(exvec[h] for h in range(H))
                for c in range(F // 16):
                    rows_v[b, e, pl.ds(c * 16, 16)] = (
                        rows_v[b, e, pl.ds(c * 16, 16)] * ss[c // 2])

            pltpu.async_copy(rows_v.at[b], acc.at[dstv_scat.at[b]],
                             ssem_r.at[b], add=True)
            pltpu.async_copy(exb.at[b], accden.at[dstv_scat.at[b]],
                             ssem_d.at[b], add=True)

        pltpu.async_copy(src_hbm.at[pl.ds(base, CH)], srcv.at[0], isem_s.at[0])
        pltpu.async_copy(dst_hbm.at[pl.ds(base, CH)], dstv.at[0], isem_d.at[0])
        issue_idx(1, 1)
        wait_idx(0)
        issue_gathers(0)

        @pl.loop(0, n_chunks // 2)
        def _(jj):
            i0 = 2 * jj
            wait_idx(1)

            @pl.when(jj > 0)
            def _():
                wait_scatters(1)
            issue_gathers(1)
            wait_gathers(0)
            copy_dstv(0)
            issue_idx(0, i0 + 2)
            compute_and_scatter(0)
            wait_idx(0)
            wait_scatters(0)
            issue_gathers(0)
            wait_gathers(1)
            copy_dstv(1)

            @pl.when(i0 + 3 < n_chunks)
            def _():
                issue_idx(1, i0 + 3)
            compute_and_scatter(1)

        if n_chunks % 2:
            wait_scatters(1)
            wait_gathers(0)
            copy_dstv(0)
            compute_and_scatter(0)
            wait_scatters(0)
        else:
            wait_scatters(0)
            wait_scatters(1)

        plsc.subcore_barrier()

        @pl.loop(0, nfull)
        def _(i):
            pltpu.sync_copy(acc.at[pl.ds(r0 + i * CH, CH)],
                            oacc_hbm.at[cid, pl.ds(r0 + i * CH, CH)])
            pltpu.sync_copy(accden.at[pl.ds(r0 + i * CH, CH)],
                            oden_hbm.at[cid, pl.ds(r0 + i * CH, CH)])

        if rem:
            pltpu.sync_copy(acc.at[pl.ds(r0 + nfull * CH, rem)],
                            oacc_hbm.at[cid, pl.ds(r0 + nfull * CH, rem)])
            pltpu.sync_copy(accden.at[pl.ds(r0 + nfull * CH, rem)],
                            oden_hbm.at[cid, pl.ds(r0 + nfull * CH, rem)])

    acc2, den2 = sc_pass(src, dst, s16, whflat)

    def comb_body(a_ref, d_ref, o_ref):
        s = a_ref[0] + a_ref[1]
        den = d_ref[0] + d_ref[1]
        r = 1.0 / (den[:, :H] + 1e-16)
        parts = [s[:, h * DH:(h + 1) * DH] * r[:, h:h + 1] for h in range(H)]
        o_ref[...] = jnp.concatenate(parts, axis=1)

    hp = pl.pallas_call(
        comb_body,
        out_shape=jax.ShapeDtypeStruct((N, F), jnp.float32),
    )(acc2, den2)
    return hp


# speedup vs baseline: 1.0514x; 1.0514x over previous
import functools

import jax
import jax.numpy as jnp
from jax import lax
from jax.experimental import pallas as pl
from jax.experimental.pallas import tpu as pltpu
from jax.experimental.pallas import tpu_sc as plsc

ALPHA = 0.2


def kernel(x, edge_index, W, Wb, a_w, a_b):
    N, D = x.shape
    H, _, DH = W.shape
    E = edge_index.shape[1]
    F = H * DH
    ROW = F + 16

    Wcat = jnp.transpose(W, (1, 0, 2)).reshape(D, F)
    bcat = Wb.reshape(1, F).astype(jnp.float32)
    eyeH = jnp.eye(H, dtype=jnp.float32)
    A_src = (a_w[:, :DH][:, :, None] * eyeH[:, None, :]).reshape(F, H)
    A_dst = (a_w[:, DH:][:, :, None] * eyeH[:, None, :]).reshape(F, H)
    Acat = jnp.concatenate([A_src, A_dst], axis=1)
    ab8 = jnp.concatenate([jnp.zeros((H,), jnp.float32), a_b]).reshape(1, 2 * H)

    def proj_body(x_ref, wc_ref, bc_ref, ac_ref, ab_ref, aug_ref, sp_ref):
        xv = x_ref[...]
        wh = lax.dot(xv, wc_ref[...], precision=lax.Precision.HIGHEST,
                     preferred_element_type=jnp.float32) + bc_ref[...]
        sp = lax.dot(wh, ac_ref[...], precision=lax.Precision.HIGHEST,
                     preferred_element_type=jnp.float32) + ab_ref[...]
        sp_ref[...] = sp
        aug_ref[...] = jnp.concatenate(
            [wh, sp[:, :H], jnp.zeros((xv.shape[0], ROW - F - H), jnp.float32)],
            axis=1)

    whaug, s8 = pl.pallas_call(
        proj_body,
        out_shape=(jax.ShapeDtypeStruct((N, ROW), jnp.float32),
                   jax.ShapeDtypeStruct((N, 2 * H), jnp.float32)),
    )(x, Wcat, bcat, Acat, ab8)

    NC, NS = 2, 16
    NW = NC * NS
    e_per_w = E // NW
    CH = 80
    n_chunks = e_per_w // CH
    rows_per_tile = N // NS
    nfull = rows_per_tile // CH
    rem = rows_per_tile - nfull * CH

    src = edge_index[0]
    dst = edge_index[1]

    mesh = plsc.VectorSubcoreMesh(core_axis_name="c", subcore_axis_name="s")

    @functools.partial(
        pl.kernel, mesh=mesh,
        compiler_params=pltpu.CompilerParams(use_tc_tiling_on_sc=False,
                                             needs_layout_passes=False),
        out_type=jax.ShapeDtypeStruct((NC, N, ROW), jnp.float32),
        scratch_types=[
            pltpu.VMEM((2, CH, ROW), jnp.float32),
            pltpu.VMEM((2, CH, 2 * H), jnp.float32),
            pltpu.VMEM((2, CH), jnp.int32),
            pltpu.VMEM((2, CH), jnp.int32),
            pltpu.VMEM((2, CH), jnp.int32),
            pltpu.VMEM_SHARED((N, 2 * H), jnp.float32),
            pltpu.VMEM_SHARED((N, ROW), jnp.float32),
            pltpu.SemaphoreType.DMA((2,)),
            pltpu.SemaphoreType.DMA((2,)),
            pltpu.SemaphoreType.DMA((2,)),
            pltpu.SemaphoreType.DMA((2,)),
            pltpu.SemaphoreType.DMA((2,)),
        ],
    )
    def sc_pass(src_hbm, dst_hbm, s8_hbm, aug_hbm, out_hbm,
                rows_v, sdst8, srcv, dstv, dstv_scat, sdst_sh, acc,
                gsem_r, gsem_d, ssem, isem_s, isem_d):
        cid = lax.axis_index("c")
        sid = lax.axis_index("s")
        wid = cid * NS + sid
        base = wid * e_per_w
        zero16 = jnp.zeros((16,), jnp.float32)

        @pl.when(sid == 0)
        def _():
            pltpu.sync_copy(s8_hbm, sdst_sh)

        @pl.loop(0, CH)
        def _(r):
            for c in range(ROW // 16):
                rows_v[0, r, pl.ds(c * 16, 16)] = zero16

        r0 = sid * rows_per_tile

        @pl.loop(0, nfull)
        def _(i):
            pltpu.sync_copy(rows_v.at[0], acc.at[pl.ds(r0 + i * CH, CH)])

        if rem:
            pltpu.sync_copy(rows_v.at[0, pl.ds(0, rem)],
                            acc.at[pl.ds(r0 + nfull * CH, rem)])
        plsc.subcore_barrier()

        def issue_idx(b, j):
            eb = base + j * CH
            pltpu.async_copy(src_hbm.at[pl.ds(eb, CH)], srcv.at[b], isem_s.at[b])
            pltpu.async_copy(dst_hbm.at[pl.ds(eb, CH)], dstv.at[b], isem_d.at[b])

        def wait_idx(b):
            pltpu.make_async_copy(src_hbm.at[pl.ds(0, CH)], srcv.at[b],
                                  isem_s.at[b]).wait()
            pltpu.make_async_copy(dst_hbm.at[pl.ds(0, CH)], dstv.at[b],
                                  isem_d.at[b]).wait()

        def issue_gathers(b):
            pltpu.async_copy(aug_hbm.at[srcv.at[b]], rows_v.at[b], gsem_r.at[b])
            pltpu.async_copy(sdst_sh.at[dstv.at[b]], sdst8.at[b], gsem_d.at[b])

        def wait_gathers(b):
            pltpu.make_async_copy(aug_hbm.at[srcv.at[b]], rows_v.at[b],
                                  gsem_r.at[b]).wait()
            pltpu.make_async_copy(sdst_sh.at[dstv.at[b]], sdst8.at[b],
                                  gsem_d.at[b]).wait()

        def wait_scatter(b):
            pltpu.make_async_copy(rows_v.at[b], acc.at[dstv_scat.at[b]],
                                  ssem.at[b]).wait()

        def copy_dstv(b):
            @pl.loop(0, CH // 16)
            def _(g):
                dstv_scat[b, pl.ds(g * 16, 16)] = dstv[b, pl.ds(g * 16, 16)]

        def compute_and_scatter(b):
            @pl.loop(0, CH // 16)
            def _(g):
                iota = lax.iota(jnp.int32, 16)
                e_vec = g * 16 + iota
                av = [plsc.load_gather(rows_v.at[b],
                                       [e_vec, jnp.full((16,), F + h, jnp.int32)])
                      for h in range(H)]
                bv = [plsc.load_gather(sdst8.at[b],
                                       [e_vec, jnp.full((16,), H + h, jnp.int32)])
                      for h in range(H)]
                zv = [a + bb for a, bb in zip(av, bv)]
                ev = [jnp.exp(jnp.maximum(z, z * ALPHA)) for z in zv]
                for h in range(H):
                    plsc.store_scatter(rows_v.at[b],
                                       [e_vec, jnp.full((16,), F + h, jnp.int32)],
                                       ev[h])

            @pl.loop(0, CH, unroll=4)
            def _(e):
                exvec = rows_v[b, e, pl.ds(F, 16)]
                ss = tuple(exvec[h] for h in range(H))
                for c in range(F // 16):
                    rows_v[b, e, pl.ds(c * 16, 16)] = (
                        rows_v[b, e, pl.ds(c * 16, 16)] * ss[c // 2])

            pltpu.async_copy(rows_v.at[b], acc.at[dstv_scat.at[b]],
                             ssem.at[b], add=True)

        issue_idx(0, 0)
        issue_idx(1, 1)
        wait_idx(0)
        issue_gathers(0)

        @pl.loop(0, n_chunks // 2)
        def _(jj):
            i0 = 2 * jj
            wait_idx(1)

            @pl.when(jj > 0)
            def _():
                wait_scatter(1)
            issue_gathers(1)
            wait_gathers(0)
            copy_dstv(0)
            issue_idx(0, i0 + 2)
            compute_and_scatter(0)
            wait_idx(0)
            wait_scatter(0)
            issue_gathers(0)
            wait_gathers(1)
            copy_dstv(1)

            @pl.when(i0 + 3 < n_chunks)
            def _():
                issue_idx(1, i0 + 3)
            compute_and_scatter(1)

        if n_chunks % 2:
            wait_scatter(1)
            wait_gathers(0)
            copy_dstv(0)
            compute_and_scatter(0)
            wait_scatter(0)
        else:
            wait_scatter(0)
            wait_scatter(1)

        plsc.subcore_barrier()

        @pl.loop(0, nfull)
        def _(i):
            pltpu.sync_copy(acc.at[pl.ds(r0 + i * CH, CH)],
                            out_hbm.at[cid, pl.ds(r0 + i * CH, CH)])

        if rem:
            pltpu.sync_copy(acc.at[pl.ds(r0 + nfull * CH, rem)],
                            out_hbm.at[cid, pl.ds(r0 + nfull * CH, rem)])

    acc2 = sc_pass(src, dst, s8, whaug)

    def comb_body(a_ref, o_ref):
        s = a_ref[0] + a_ref[1]
        den = s[:, F:F + H]
        r = 1.0 / (den + 1e-16)
        parts = [s[:, h * DH:(h + 1) * DH] * r[:, h:h + 1] for h in range(H)]
        o_ref[...] = jnp.concatenate(parts, axis=1)

    hp = pl.pallas_call(
        comb_body,
        out_shape=jax.ShapeDtypeStruct((N, F), jnp.float32),
    )(acc2)
    return hp


# speedup vs baseline: 1.0972x; 1.0435x over previous
import functools

import jax
import jax.numpy as jnp
from jax import lax
from jax.experimental import pallas as pl
from jax.experimental.pallas import tpu as pltpu
from jax.experimental.pallas import tpu_sc as plsc

ALPHA = 0.2


def kernel(x, edge_index, W, Wb, a_w, a_b):
    N, D = x.shape
    H, _, DH = W.shape
    E = edge_index.shape[1]
    F = H * DH
    ROW = F + 16

    Wcat = jnp.transpose(W, (1, 0, 2)).reshape(D, F)
    bcat = Wb.reshape(1, F).astype(jnp.float32)
    eyeH = jnp.eye(H, dtype=jnp.float32)
    A_src = (a_w[:, :DH][:, :, None] * eyeH[:, None, :]).reshape(F, H)
    A_dst = (a_w[:, DH:][:, :, None] * eyeH[:, None, :]).reshape(F, H)
    Acat = jnp.concatenate([A_src, A_dst], axis=1)
    ab8 = jnp.concatenate([jnp.zeros((H,), jnp.float32), a_b]).reshape(1, 2 * H)

    ab4 = a_b.reshape(1, H)

    def proj_body(x_ref, wc_ref, bc_ref, as_ref, ad_ref, ab_ref,
                  aug_ref, sp_ref):
        xv = x_ref[...]
        wh = lax.dot(xv, wc_ref[...], precision=lax.Precision.HIGHEST,
                     preferred_element_type=jnp.float32) + bc_ref[...]
        ssrc = lax.dot(wh, as_ref[...], precision=lax.Precision.HIGHEST,
                       preferred_element_type=jnp.float32)
        sdst = lax.dot(wh, ad_ref[...], precision=lax.Precision.HIGHEST,
                       preferred_element_type=jnp.float32) + ab_ref[...]
        zpad = jnp.zeros((xv.shape[0], ROW - F - H), jnp.float32)
        sp_ref[...] = jnp.concatenate(
            [sdst, jnp.zeros((xv.shape[0], 12), jnp.float32)], axis=1)
        aug_ref[...] = jnp.concatenate([wh, ssrc, zpad], axis=1)

    whaug, s16 = pl.pallas_call(
        proj_body,
        out_shape=(jax.ShapeDtypeStruct((N, ROW), jnp.float32),
                   jax.ShapeDtypeStruct((N, 16), jnp.float32)),
    )(x, Wcat, bcat, A_src, A_dst, ab4)

    NC, NS = 2, 16
    NW = NC * NS
    e_per_w = E // NW
    CH = 80
    n_chunks = e_per_w // CH
    rows_per_tile = N // NS
    nfull = rows_per_tile // CH
    rem = rows_per_tile - nfull * CH

    src = edge_index[0]
    dst = edge_index[1]

    mesh = plsc.VectorSubcoreMesh(core_axis_name="c", subcore_axis_name="s")

    @functools.partial(
        pl.kernel, mesh=mesh,
        compiler_params=pltpu.CompilerParams(use_tc_tiling_on_sc=False,
                                             needs_layout_passes=False),
        out_type=jax.ShapeDtypeStruct((NC, N, ROW), jnp.float32),
        scratch_types=[
            pltpu.VMEM((3, CH, ROW), jnp.float32),
            pltpu.VMEM((3, CH, 16), jnp.float32),
            pltpu.VMEM((3, CH), jnp.int32),
            pltpu.VMEM((3, CH), jnp.int32),
            pltpu.VMEM((3, CH), jnp.int32),
            pltpu.VMEM_SHARED((N, ROW), jnp.float32),
            pltpu.SemaphoreType.DMA((3,)),
            pltpu.SemaphoreType.DMA((3,)),
            pltpu.SemaphoreType.DMA((3,)),
            pltpu.SemaphoreType.DMA((3,)),
            pltpu.SemaphoreType.DMA((3,)),
        ],
    )
    def sc_pass(src_hbm, dst_hbm, s16_hbm, aug_hbm, out_hbm,
                rows_v, sdst16, srcv, dstv, dstv_scat, acc,
                gsem_r, gsem_d, ssem, isem_s, isem_d):
        cid = lax.axis_index("c")
        sid = lax.axis_index("s")
        wid = cid * NS + sid
        base = wid * e_per_w
        zero16 = jnp.zeros((16,), jnp.float32)

        @pl.loop(0, CH)
        def _(r):
            for c in range(ROW // 16):
                rows_v[0, r, pl.ds(c * 16, 16)] = zero16

        r0 = sid * rows_per_tile

        @pl.loop(0, nfull)
        def _(i):
            pltpu.sync_copy(rows_v.at[0], acc.at[pl.ds(r0 + i * CH, CH)])

        if rem:
            pltpu.sync_copy(rows_v.at[0, pl.ds(0, rem)],
                            acc.at[pl.ds(r0 + nfull * CH, rem)])
        plsc.subcore_barrier()

        def issue_idx(b, j):
            eb = base + j * CH
            pltpu.async_copy(src_hbm.at[pl.ds(eb, CH)], srcv.at[b], isem_s.at[b])
            pltpu.async_copy(dst_hbm.at[pl.ds(eb, CH)], dstv.at[b], isem_d.at[b])

        def wait_idx(b):
            pltpu.make_async_copy(src_hbm.at[pl.ds(0, CH)], srcv.at[b],
                                  isem_s.at[b]).wait()
            pltpu.make_async_copy(dst_hbm.at[pl.ds(0, CH)], dstv.at[b],
                                  isem_d.at[b]).wait()

        def issue_gathers(b):
            pltpu.async_copy(aug_hbm.at[srcv.at[b]], rows_v.at[b], gsem_r.at[b])
            pltpu.async_copy(s16_hbm.at[dstv.at[b]], sdst16.at[b], gsem_d.at[b])

        def wait_gathers(b):
            pltpu.make_async_copy(aug_hbm.at[srcv.at[b]], rows_v.at[b],
                                  gsem_r.at[b]).wait()
            pltpu.make_async_copy(s16_hbm.at[dstv.at[b]], sdst16.at[b],
                                  gsem_d.at[b]).wait()

        def wait_scatter(b):
            pltpu.make_async_copy(rows_v.at[b], acc.at[dstv_scat.at[b]],
                                  ssem.at[b]).wait()

        def copy_dstv(b):
            @pl.loop(0, CH // 16)
            def _(g):
                dstv_scat[b, pl.ds(g * 16, 16)] = dstv[b, pl.ds(g * 16, 16)]

        def compute_and_scatter(b):
            @pl.loop(0, CH // 16)
            def _(g):
                iota = lax.iota(jnp.int32, 16)
                e_vec = g * 16 + iota
                av = [plsc.load_gather(rows_v.at[b],
                                       [e_vec, jnp.full((16,), F + h, jnp.int32)])
                      for h in range(H)]
                bv = [plsc.load_gather(sdst16.at[b],
                                       [e_vec, jnp.full((16,), h, jnp.int32)])
                      for h in range(H)]
                zv = [a + bb for a, bb in zip(av, bv)]
                ev = [jnp.exp(jnp.maximum(z, z * ALPHA)) for z in zv]
                for h in range(H):
                    plsc.store_scatter(rows_v.at[b],
                                       [e_vec, jnp.full((16,), F + h, jnp.int32)],
                                       ev[h])

            @pl.loop(0, CH, unroll=4)
            def _(e):
                exvec = rows_v[b, e, pl.ds(F, 16)]
                ss = tuple(exvec[h] for h in range(H))
                for c in range(F // 16):
                    rows_v[b, e, pl.ds(c * 16, 16)] = (
                        rows_v[b, e, pl.ds(c * 16, 16)] * ss[c // 2])

            pltpu.async_copy(rows_v.at[b], acc.at[dstv_scat.at[b]],
                             ssem.at[b], add=True)

        def chunk_body(q, j_plus_2, guard_scatter):
            q1 = (q + 1) % 3
            wait_idx(q1)
            if guard_scatter is None:
                wait_scatter(q1)
            elif guard_scatter is not False:
                @pl.when(guard_scatter)
                def _():
                    wait_scatter(q1)
            issue_gathers(q1)
            wait_gathers(q)
            copy_dstv(q)
            if j_plus_2 is not None:
                issue_idx((q + 2) % 3, j_plus_2)
            compute_and_scatter(q)

        issue_idx(0, 0)
        issue_idx(1, 1)
        wait_idx(0)
        issue_gathers(0)

        n_triples = n_chunks // 3

        @pl.loop(0, n_triples)
        def _(t):
            j0 = 3 * t
            chunk_body(0, j0 + 2, t > 0)
            chunk_body(1, j0 + 3, t > 0)
            chunk_body(2, j0 + 4, None)

        chunk_body(0, None, None)
        q = 1
        wait_scatter(2)
        wait_gathers(q)
        copy_dstv(q)
        compute_and_scatter(q)
        wait_scatter(0)
        wait_scatter(1)

        plsc.subcore_barrier()

        @pl.loop(0, nfull)
        def _(i):
            pltpu.sync_copy(acc.at[pl.ds(r0 + i * CH, CH)],
                            out_hbm.at[cid, pl.ds(r0 + i * CH, CH)])

        if rem:
            pltpu.sync_copy(acc.at[pl.ds(r0 + nfull * CH, rem)],
                            out_hbm.at[cid, pl.ds(r0 + nfull * CH, rem)])

    acc2 = sc_pass(src, dst, s16, whaug)

    def comb_body(a_ref, o_ref):
        s = a_ref[0] + a_ref[1]
        den = s[:, F:F + H]
        r = 1.0 / (den + 1e-16)
        parts = [s[:, h * DH:(h + 1) * DH] * r[:, h:h + 1] for h in range(H)]
        o_ref[...] = jnp.concatenate(parts, axis=1)

    hp = pl.pallas_call(
        comb_body,
        out_shape=jax.ShapeDtypeStruct((N, F), jnp.float32),
    )(acc2)
    return hp


# speedup vs baseline: 1.1067x; 1.0087x over previous
import functools

import jax
import jax.numpy as jnp
from jax import lax
from jax.experimental import pallas as pl
from jax.experimental.pallas import tpu as pltpu
from jax.experimental.pallas import tpu_sc as plsc

ALPHA = 0.2


def kernel(x, edge_index, W, Wb, a_w, a_b):
    N, D = x.shape
    H, _, DH = W.shape
    E = edge_index.shape[1]
    F = H * DH
    ROW = F + 16

    Wcat = jnp.transpose(W, (1, 0, 2)).reshape(D, F)
    bcat = Wb.reshape(1, F).astype(jnp.float32)
    eyeH = jnp.eye(H, dtype=jnp.float32)
    A_src = (a_w[:, :DH][:, :, None] * eyeH[:, None, :]).reshape(F, H)
    A_dst = (a_w[:, DH:][:, :, None] * eyeH[:, None, :]).reshape(F, H)
    Acat = jnp.concatenate([A_src, A_dst], axis=1)
    ab8 = jnp.concatenate([jnp.zeros((H,), jnp.float32), a_b]).reshape(1, 2 * H)

    ab4 = a_b.reshape(1, H)

    def proj_body(x_ref, wc_ref, bc_ref, as_ref, ad_ref, ab_ref,
                  aug_ref, sp_ref):
        xv = x_ref[...]
        wh = lax.dot(xv, wc_ref[...], precision=lax.Precision.HIGHEST,
                     preferred_element_type=jnp.float32) + bc_ref[...]
        ssrc = lax.dot(wh, as_ref[...], precision=lax.Precision.HIGHEST,
                       preferred_element_type=jnp.float32)
        sdst = lax.dot(wh, ad_ref[...], precision=lax.Precision.HIGHEST,
                       preferred_element_type=jnp.float32) + ab_ref[...]
        zpad = jnp.zeros((xv.shape[0], ROW - F - H), jnp.float32)
        sp_ref[...] = jnp.concatenate(
            [sdst, jnp.zeros((xv.shape[0], 12), jnp.float32)], axis=1)
        aug_ref[...] = jnp.concatenate([wh, ssrc, zpad], axis=1)

    whaug, s16 = pl.pallas_call(
        proj_body,
        out_shape=(jax.ShapeDtypeStruct((N, ROW), jnp.float32),
                   jax.ShapeDtypeStruct((N, 16), jnp.float32)),
    )(x, Wcat, bcat, A_src, A_dst, ab4)

    NC, NS = 2, 16
    NW = NC * NS
    e_per_w = E // NW
    CH = 80
    n_chunks = e_per_w // CH
    rows_per_tile = N // NS
    nfull = rows_per_tile // CH
    rem = rows_per_tile - nfull * CH

    src = edge_index[0]
    dst = edge_index[1]

    mesh = plsc.VectorSubcoreMesh(core_axis_name="c", subcore_axis_name="s")

    @functools.partial(
        pl.kernel, mesh=mesh,
        compiler_params=pltpu.CompilerParams(use_tc_tiling_on_sc=False,
                                             needs_layout_passes=False),
        out_type=jax.ShapeDtypeStruct((NC, N, ROW), jnp.float32),
        scratch_types=[
            pltpu.VMEM((3, CH, ROW), jnp.float32),
            pltpu.VMEM((3, CH, 16), jnp.float32),
            pltpu.VMEM((3, CH), jnp.int32),
            pltpu.VMEM((3, CH), jnp.int32),
            pltpu.VMEM((3, CH), jnp.int32),
            pltpu.VMEM_SHARED((N, ROW), jnp.float32),
            pltpu.SemaphoreType.DMA((3,)),
            pltpu.SemaphoreType.DMA((3,)),
            pltpu.SemaphoreType.DMA((3,)),
            pltpu.SemaphoreType.DMA((3,)),
            pltpu.SemaphoreType.DMA((3,)),
        ],
    )
    def sc_pass(src_hbm, dst_hbm, s16_hbm, aug_hbm, out_hbm,
                rows_v, sdst16, srcv, dstv, dstv_scat, acc,
                gsem_r, gsem_d, ssem, isem_s, isem_d):
        cid = lax.axis_index("c")
        sid = lax.axis_index("s")
        wid = cid * NS + sid
        base = wid * e_per_w
        zero16 = jnp.zeros((16,), jnp.float32)
        r0 = sid * rows_per_tile

        def issue_idx(b, j):
            eb = base + j * CH
            pltpu.async_copy(src_hbm.at[pl.ds(eb, CH)], srcv.at[b], isem_s.at[b])
            pltpu.async_copy(dst_hbm.at[pl.ds(eb, CH)], dstv.at[b], isem_d.at[b])

        def wait_idx(b):
            pltpu.make_async_copy(src_hbm.at[pl.ds(0, CH)], srcv.at[b],
                                  isem_s.at[b]).wait()
            pltpu.make_async_copy(dst_hbm.at[pl.ds(0, CH)], dstv.at[b],
                                  isem_d.at[b]).wait()

        def issue_gathers(b):
            pltpu.async_copy(aug_hbm.at[srcv.at[b]], rows_v.at[b], gsem_r.at[b])
            pltpu.async_copy(s16_hbm.at[dstv.at[b]], sdst16.at[b], gsem_d.at[b])

        def wait_gathers(b):
            pltpu.make_async_copy(aug_hbm.at[srcv.at[b]], rows_v.at[b],
                                  gsem_r.at[b]).wait()
            pltpu.make_async_copy(s16_hbm.at[dstv.at[b]], sdst16.at[b],
                                  gsem_d.at[b]).wait()

        def wait_scatter(b):
            pltpu.make_async_copy(rows_v.at[b], acc.at[dstv_scat.at[b]],
                                  ssem.at[b]).wait()

        def copy_dstv(b):
            @pl.loop(0, CH // 16)
            def _(g):
                dstv_scat[b, pl.ds(g * 16, 16)] = dstv[b, pl.ds(g * 16, 16)]

        def compute_and_scatter(b):
            @pl.loop(0, CH // 16)
            def _(g):
                iota = lax.iota(jnp.int32, 16)
                e_vec = g * 16 + iota
                av = [plsc.load_gather(rows_v.at[b],
                                       [e_vec, jnp.full((16,), F + h, jnp.int32)])
                      for h in range(H)]
                bv = [plsc.load_gather(sdst16.at[b],
                                       [e_vec, jnp.full((16,), h, jnp.int32)])
                      for h in range(H)]
                zv = [a + bb for a, bb in zip(av, bv)]
                ev = [jnp.exp(jnp.maximum(z, z * ALPHA)) for z in zv]
                for h in range(H):
                    plsc.store_scatter(rows_v.at[b],
                                       [e_vec, jnp.full((16,), F + h, jnp.int32)],
                                       ev[h])

            @pl.loop(0, CH, unroll=4)
            def _(e):
                exvec = rows_v[b, e, pl.ds(F, 16)]
                ss = tuple(exvec[h] for h in range(H))
                for c in range(F // 16):
                    rows_v[b, e, pl.ds(c * 16, 16)] = (
                        rows_v[b, e, pl.ds(c * 16, 16)] * ss[c // 2])

            pltpu.async_copy(rows_v.at[b], acc.at[dstv_scat.at[b]],
                             ssem.at[b], add=True)

        def chunk_body(q, j_plus_2, guard_scatter):
            q1 = (q + 1) % 3
            wait_idx(q1)
            if guard_scatter is None:
                wait_scatter(q1)
            elif guard_scatter is not False:
                @pl.when(guard_scatter)
                def _():
                    wait_scatter(q1)
            issue_gathers(q1)
            wait_gathers(q)
            copy_dstv(q)
            if j_plus_2 is not None:
                issue_idx((q + 2) % 3, j_plus_2)
            compute_and_scatter(q)

        issue_idx(0, 0)
        issue_idx(1, 1)

        @pl.loop(0, CH)
        def _(r):
            for c in range(ROW // 16):
                rows_v[1, r, pl.ds(c * 16, 16)] = zero16

        wait_idx(0)
        issue_gathers(0)

        @pl.loop(0, nfull)
        def _(i):
            pltpu.async_copy(rows_v.at[1], acc.at[pl.ds(r0 + i * CH, CH)],
                             gsem_d.at[1])

        if rem:
            pltpu.async_copy(rows_v.at[1, pl.ds(0, rem)],
                             acc.at[pl.ds(r0 + nfull * CH, rem)], gsem_d.at[1])

        @pl.loop(0, nfull)
        def _(i):
            pltpu.make_async_copy(rows_v.at[1], acc.at[pl.ds(r0 + i * CH, CH)],
                                  gsem_d.at[1]).wait()

        if rem:
            pltpu.make_async_copy(rows_v.at[1, pl.ds(0, rem)],
                                  acc.at[pl.ds(r0 + nfull * CH, rem)],
                                  gsem_d.at[1]).wait()
        plsc.subcore_barrier()

        n_triples = n_chunks // 3

        @pl.loop(0, n_triples)
        def _(t):
            j0 = 3 * t
            chunk_body(0, j0 + 2, t > 0)
            chunk_body(1, j0 + 3, t > 0)
            chunk_body(2, j0 + 4, None)

        chunk_body(0, None, None)
        q = 1
        wait_scatter(2)
        wait_gathers(q)
        copy_dstv(q)
        compute_and_scatter(q)
        wait_scatter(0)
        wait_scatter(1)

        plsc.subcore_barrier()

        @pl.loop(0, nfull)
        def _(i):
            pltpu.async_copy(acc.at[pl.ds(r0 + i * CH, CH)],
                             out_hbm.at[cid, pl.ds(r0 + i * CH, CH)],
                             gsem_r.at[0])

        if rem:
            pltpu.async_copy(acc.at[pl.ds(r0 + nfull * CH, rem)],
                             out_hbm.at[cid, pl.ds(r0 + nfull * CH, rem)],
                             gsem_r.at[0])

        @pl.loop(0, nfull)
        def _(i):
            pltpu.make_async_copy(acc.at[pl.ds(r0 + i * CH, CH)],
                                  out_hbm.at[cid, pl.ds(r0 + i * CH, CH)],
                                  gsem_r.at[0]).wait()

        if rem:
            pltpu.make_async_copy(acc.at[pl.ds(r0 + nfull * CH, rem)],
                                  out_hbm.at[cid, pl.ds(r0 + nfull * CH, rem)],
                                  gsem_r.at[0]).wait()

    acc2 = sc_pass(src, dst, s16, whaug)

    def comb_body(a_ref, o_ref):
        s = a_ref[0] + a_ref[1]
        den = s[:, F:F + H]
        r = 1.0 / (den + 1e-16)
        parts = [s[:, h * DH:(h + 1) * DH] * r[:, h:h + 1] for h in range(H)]
        o_ref[...] = jnp.concatenate(parts, axis=1)

    hp = pl.pallas_call(
        comb_body,
        out_shape=jax.ShapeDtypeStruct((N, F), jnp.float32),
    )(acc2)
    return hp


# speedup vs baseline: 1.1719x; 1.0589x over previous
import functools

import jax
import jax.numpy as jnp
from jax import lax
from jax.experimental import pallas as pl
from jax.experimental.pallas import tpu as pltpu
from jax.experimental.pallas import tpu_sc as plsc

ALPHA = 0.2


def kernel(x, edge_index, W, Wb, a_w, a_b):
    N, D = x.shape
    H, _, DH = W.shape
    E = edge_index.shape[1]
    F = H * DH
    ROW = F + 16

    Wcat = jnp.transpose(W, (1, 0, 2)).reshape(D, F)
    bcat = Wb.reshape(1, F).astype(jnp.float32)
    eyeH = jnp.eye(H, dtype=jnp.float32)
    A_src = (a_w[:, :DH][:, :, None] * eyeH[:, None, :]).reshape(F, H)
    A_dst = (a_w[:, DH:][:, :, None] * eyeH[:, None, :]).reshape(F, H)
    Acat = jnp.concatenate([A_src, A_dst], axis=1)
    ab8 = jnp.concatenate([jnp.zeros((H,), jnp.float32), a_b]).reshape(1, 2 * H)

    ab4 = a_b.reshape(1, H)

    def proj_body(x_ref, wc_ref, bc_ref, as_ref, ad_ref, ab_ref, ei_ref,
                  aug_ref, sp_ref, src_ref, dst_ref):
        xv = x_ref[...]
        wh = lax.dot(xv, wc_ref[...], precision=lax.Precision.HIGHEST,
                     preferred_element_type=jnp.float32) + bc_ref[...]
        ssrc = lax.dot(wh, as_ref[...], precision=lax.Precision.HIGHEST,
                       preferred_element_type=jnp.float32)
        sdst = lax.dot(wh, ad_ref[...], precision=lax.Precision.HIGHEST,
                       preferred_element_type=jnp.float32) + ab_ref[...]
        zpad = jnp.zeros((xv.shape[0], ROW - F - H), jnp.float32)
        sp_ref[...] = jnp.concatenate(
            [sdst, jnp.zeros((xv.shape[0], 12), jnp.float32)], axis=1)
        aug_ref[...] = jnp.concatenate([wh, ssrc, zpad], axis=1)
        src_ref[...] = ei_ref[0]
        dst_ref[...] = ei_ref[1]

    whaug, s16, src, dst = pl.pallas_call(
        proj_body,
        out_shape=(jax.ShapeDtypeStruct((N, ROW), jnp.float32),
                   jax.ShapeDtypeStruct((N, 16), jnp.float32),
                   jax.ShapeDtypeStruct((E,), jnp.int32),
                   jax.ShapeDtypeStruct((E,), jnp.int32)),
    )(x, Wcat, bcat, A_src, A_dst, ab4, edge_index)

    NC, NS = 2, 16
    NW = NC * NS
    e_per_w = E // NW
    CH = 80
    n_chunks = e_per_w // CH
    rows_per_tile = N // NS
    nfull = rows_per_tile // CH
    rem = rows_per_tile - nfull * CH

    mesh = plsc.VectorSubcoreMesh(core_axis_name="c", subcore_axis_name="s")

    @functools.partial(
        pl.kernel, mesh=mesh,
        compiler_params=pltpu.CompilerParams(use_tc_tiling_on_sc=False,
                                             needs_layout_passes=False),
        out_type=jax.ShapeDtypeStruct((NC, N, ROW), jnp.float32),
        scratch_types=[
            pltpu.VMEM((3, CH, ROW), jnp.float32),
            pltpu.VMEM((3, CH, 16), jnp.float32),
            pltpu.VMEM((3, CH), jnp.int32),
            pltpu.VMEM((3, CH), jnp.int32),
            pltpu.VMEM((3, CH), jnp.int32),
            pltpu.VMEM_SHARED((N, ROW), jnp.float32),
            pltpu.SemaphoreType.DMA((3,)),
            pltpu.SemaphoreType.DMA((3,)),
            pltpu.SemaphoreType.DMA((3,)),
            pltpu.SemaphoreType.DMA((3,)),
            pltpu.SemaphoreType.DMA((3,)),
        ],
    )
    def sc_pass(src_hbm, dst_hbm, s16_hbm, aug_hbm, out_hbm,
                rows_v, sdst16, srcv, dstv, dstv_scat, acc,
                gsem_r, gsem_d, ssem, isem_s, isem_d):
        cid = lax.axis_index("c")
        sid = lax.axis_index("s")
        wid = cid * NS + sid
        base = wid * e_per_w
        zero16 = jnp.zeros((16,), jnp.float32)
        r0 = sid * rows_per_tile

        def issue_idx(b, j):
            eb = base + j * CH
            pltpu.async_copy(src_hbm.at[pl.ds(eb, CH)], srcv.at[b], isem_s.at[b])
            pltpu.async_copy(dst_hbm.at[pl.ds(eb, CH)], dstv.at[b], isem_d.at[b])

        def wait_idx(b):
            pltpu.make_async_copy(src_hbm.at[pl.ds(0, CH)], srcv.at[b],
                                  isem_s.at[b]).wait()
            pltpu.make_async_copy(dst_hbm.at[pl.ds(0, CH)], dstv.at[b],
                                  isem_d.at[b]).wait()

        def issue_gathers(b):
            pltpu.async_copy(aug_hbm.at[srcv.at[b]], rows_v.at[b], gsem_r.at[b])
            pltpu.async_copy(s16_hbm.at[dstv.at[b]], sdst16.at[b], gsem_d.at[b])

        def wait_gathers(b):
            pltpu.make_async_copy(aug_hbm.at[srcv.at[b]], rows_v.at[b],
                                  gsem_r.at[b]).wait()
            pltpu.make_async_copy(s16_hbm.at[dstv.at[b]], sdst16.at[b],
                                  gsem_d.at[b]).wait()

        def wait_scatter(b):
            pltpu.make_async_copy(rows_v.at[b], acc.at[dstv_scat.at[b]],
                                  ssem.at[b]).wait()

        def copy_dstv(b):
            @pl.loop(0, CH // 16)
            def _(g):
                dstv_scat[b, pl.ds(g * 16, 16)] = dstv[b, pl.ds(g * 16, 16)]

        def compute_and_scatter(b):
            @pl.loop(0, CH // 16)
            def _(g):
                iota = lax.iota(jnp.int32, 16)
                e_vec = g * 16 + iota
                av = [plsc.load_gather(rows_v.at[b],
                                       [e_vec, jnp.full((16,), F + h, jnp.int32)])
                      for h in range(H)]
                bv = [plsc.load_gather(sdst16.at[b],
                                       [e_vec, jnp.full((16,), h, jnp.int32)])
                      for h in range(H)]
                zv = [a + bb for a, bb in zip(av, bv)]
                ev = [jnp.exp(jnp.maximum(z, z * ALPHA)) for z in zv]
                for h in range(H):
                    plsc.store_scatter(rows_v.at[b],
                                       [e_vec, jnp.full((16,), F + h, jnp.int32)],
                                       ev[h])

            @pl.loop(0, CH, unroll=4)
            def _(e):
                exvec = rows_v[b, e, pl.ds(F, 16)]
                ss = tuple(exvec[h] for h in range(H))
                for c in range(F // 16):
                    rows_v[b, e, pl.ds(c * 16, 16)] = (
                        rows_v[b, e, pl.ds(c * 16, 16)] * ss[c // 2])

            pltpu.async_copy(rows_v.at[b], acc.at[dstv_scat.at[b]],
                             ssem.at[b], add=True)

        def chunk_body(q, j_plus_2, guard_scatter):
            q1 = (q + 1) % 3
            wait_idx(q1)
            if guard_scatter is None:
                wait_scatter(q1)
            elif guard_scatter is not False:
                @pl.when(guard_scatter)
                def _():
                    wait_scatter(q1)
            issue_gathers(q1)
            wait_gathers(q)
            copy_dstv(q)
            if j_plus_2 is not None:
                issue_idx((q + 2) % 3, j_plus_2)
            compute_and_scatter(q)

        issue_idx(0, 0)
        issue_idx(1, 1)

        @pl.loop(0, CH)
        def _(r):
            for c in range(ROW // 16):
                rows_v[1, r, pl.ds(c * 16, 16)] = zero16

        wait_idx(0)
        issue_gathers(0)

        @pl.loop(0, nfull)
        def _(i):
            pltpu.async_copy(rows_v.at[1], acc.at[pl.ds(r0 + i * CH, CH)],
                             gsem_d.at[1])

        if rem:
            pltpu.async_copy(rows_v.at[1, pl.ds(0, rem)],
                             acc.at[pl.ds(r0 + nfull * CH, rem)], gsem_d.at[1])

        @pl.loop(0, nfull)
        def _(i):
            pltpu.make_async_copy(rows_v.at[1], acc.at[pl.ds(r0 + i * CH, CH)],
                                  gsem_d.at[1]).wait()

        if rem:
            pltpu.make_async_copy(rows_v.at[1, pl.ds(0, rem)],
                                  acc.at[pl.ds(r0 + nfull * CH, rem)],
                                  gsem_d.at[1]).wait()
        plsc.subcore_barrier()

        n_triples = n_chunks // 3

        @pl.loop(0, n_triples)
        def _(t):
            j0 = 3 * t
            chunk_body(0, j0 + 2, t > 0)
            chunk_body(1, j0 + 3, t > 0)
            chunk_body(2, j0 + 4, None)

        chunk_body(0, None, None)
        q = 1
        wait_scatter(2)
        wait_gathers(q)
        copy_dstv(q)
        compute_and_scatter(q)
        wait_scatter(0)
        wait_scatter(1)

        plsc.subcore_barrier()

        @pl.loop(0, nfull)
        def _(i):
            pltpu.async_copy(acc.at[pl.ds(r0 + i * CH, CH)],
                             out_hbm.at[cid, pl.ds(r0 + i * CH, CH)],
                             gsem_r.at[0])

        if rem:
            pltpu.async_copy(acc.at[pl.ds(r0 + nfull * CH, rem)],
                             out_hbm.at[cid, pl.ds(r0 + nfull * CH, rem)],
                             gsem_r.at[0])

        @pl.loop(0, nfull)
        def _(i):
            pltpu.make_async_copy(acc.at[pl.ds(r0 + i * CH, CH)],
                                  out_hbm.at[cid, pl.ds(r0 + i * CH, CH)],
                                  gsem_r.at[0]).wait()

        if rem:
            pltpu.make_async_copy(acc.at[pl.ds(r0 + nfull * CH, rem)],
                                  out_hbm.at[cid, pl.ds(r0 + nfull * CH, rem)],
                                  gsem_r.at[0]).wait()

    acc2 = sc_pass(src, dst, s16, whaug)

    def comb_body(a_ref, o_ref):
        s = a_ref[0] + a_ref[1]
        den = s[:, F:F + H]
        r = 1.0 / (den + 1e-16)
        parts = [s[:, h * DH:(h + 1) * DH] * r[:, h:h + 1] for h in range(H)]
        o_ref[...] = jnp.concatenate(parts, axis=1)

    Bn = 1000
    hp = pl.pallas_call(
        comb_body,
        grid=(N // Bn,),
        in_specs=[pl.BlockSpec((NC, Bn, ROW), lambda i: (0, i, 0))],
        out_specs=pl.BlockSpec((Bn, F), lambda i: (i, 0)),
        out_shape=jax.ShapeDtypeStruct((N, F), jnp.float32),
    )(acc2)
    return hp


# speedup vs baseline: 1.2983x; 1.1079x over previous
import functools

import jax
import jax.numpy as jnp
from jax import lax
from jax.experimental import pallas as pl
from jax.experimental.pallas import tpu as pltpu
from jax.experimental.pallas import tpu_sc as plsc

ALPHA = 0.2


def kernel(x, edge_index, W, Wb, a_w, a_b):
    N, D = x.shape
    H, _, DH = W.shape
    E = edge_index.shape[1]
    F = H * DH
    ROW = F + 16

    Wcat = jnp.transpose(W, (1, 0, 2)).reshape(D, F)
    bcat = Wb.reshape(1, F).astype(jnp.float32)
    eyeH = jnp.eye(H, dtype=jnp.float32)
    A_src = (a_w[:, :DH][:, :, None] * eyeH[:, None, :]).reshape(F, H)
    A_dst = (a_w[:, DH:][:, :, None] * eyeH[:, None, :]).reshape(F, H)
    Acat = jnp.concatenate([A_src, A_dst], axis=1)
    ab8 = jnp.concatenate([jnp.zeros((H,), jnp.float32), a_b]).reshape(1, 2 * H)

    ab4 = a_b.reshape(1, H)

    def proj_body(x_ref, wc_ref, bc_ref, as_ref, ad_ref, ab_ref, ei_ref,
                  aug_ref, sp_ref, src_ref, dst_ref):
        xv = x_ref[...]
        wh = lax.dot(xv, wc_ref[...], preferred_element_type=jnp.float32) + bc_ref[...]
        ssrc = lax.dot(wh, as_ref[...],
                       preferred_element_type=jnp.float32)
        sdst = lax.dot(wh, ad_ref[...],
                       preferred_element_type=jnp.float32) + ab_ref[...]
        zpad = jnp.zeros((xv.shape[0], ROW - F - H), jnp.float32)
        sp_ref[...] = jnp.concatenate(
            [sdst, jnp.zeros((xv.shape[0], 12), jnp.float32)], axis=1)
        aug_ref[...] = jnp.concatenate([wh, ssrc, zpad], axis=1)
        src_ref[...] = ei_ref[0]
        dst_ref[...] = ei_ref[1]

    whaug, s16, src, dst = pl.pallas_call(
        proj_body,
        out_shape=(jax.ShapeDtypeStruct((N, ROW), jnp.float32),
                   jax.ShapeDtypeStruct((N, 16), jnp.float32),
                   jax.ShapeDtypeStruct((E,), jnp.int32),
                   jax.ShapeDtypeStruct((E,), jnp.int32)),
    )(x, Wcat, bcat, A_src, A_dst, ab4, edge_index)

    NC, NS = 2, 16
    NW = NC * NS
    e_per_w = E // NW
    CH = 80
    n_chunks = e_per_w // CH
    rows_per_tile = N // NS
    nfull = rows_per_tile // CH
    rem = rows_per_tile - nfull * CH

    mesh = plsc.VectorSubcoreMesh(core_axis_name="c", subcore_axis_name="s")

    @functools.partial(
        pl.kernel, mesh=mesh,
        compiler_params=pltpu.CompilerParams(use_tc_tiling_on_sc=False,
                                             needs_layout_passes=False),
        out_type=(jax.ShapeDtypeStruct((NC, N, F), jnp.float32),
                  jax.ShapeDtypeStruct((NC, N, 16), jnp.float32)),
        scratch_types=[
            pltpu.VMEM((3, CH, ROW), jnp.float32),
            pltpu.VMEM((3, CH, 16), jnp.float32),
            pltpu.VMEM((3, CH), jnp.int32),
            pltpu.VMEM((3, CH), jnp.int32),
            pltpu.VMEM((3, CH), jnp.int32),
            pltpu.VMEM_SHARED((N, ROW), jnp.float32),
            pltpu.SemaphoreType.DMA((3,)),
            pltpu.SemaphoreType.DMA((3,)),
            pltpu.SemaphoreType.DMA((3,)),
            pltpu.SemaphoreType.DMA((3,)),
            pltpu.SemaphoreType.DMA((3,)),
        ],
    )
    def sc_pass(src_hbm, dst_hbm, s16_hbm, aug_hbm, outf_hbm, outd_hbm,
                rows_v, sdst16, srcv, dstv, dstv_scat, acc,
                gsem_r, gsem_d, ssem, isem_s, isem_d):
        cid = lax.axis_index("c")
        sid = lax.axis_index("s")
        wid = cid * NS + sid
        base = wid * e_per_w
        zero16 = jnp.zeros((16,), jnp.float32)
        r0 = sid * rows_per_tile

        def issue_idx(b, j):
            eb = base + j * CH
            pltpu.async_copy(src_hbm.at[pl.ds(eb, CH)], srcv.at[b], isem_s.at[b])
            pltpu.async_copy(dst_hbm.at[pl.ds(eb, CH)], dstv.at[b], isem_d.at[b])

        def wait_idx(b):
            pltpu.make_async_copy(src_hbm.at[pl.ds(0, CH)], srcv.at[b],
                                  isem_s.at[b]).wait()
            pltpu.make_async_copy(dst_hbm.at[pl.ds(0, CH)], dstv.at[b],
                                  isem_d.at[b]).wait()

        def issue_gathers(b):
            pltpu.async_copy(aug_hbm.at[srcv.at[b]], rows_v.at[b], gsem_r.at[b])
            pltpu.async_copy(s16_hbm.at[dstv.at[b]], sdst16.at[b], gsem_d.at[b])

        def wait_gathers(b):
            pltpu.make_async_copy(aug_hbm.at[srcv.at[b]], rows_v.at[b],
                                  gsem_r.at[b]).wait()
            pltpu.make_async_copy(s16_hbm.at[dstv.at[b]], sdst16.at[b],
                                  gsem_d.at[b]).wait()

        def wait_scatter(b):
            pltpu.make_async_copy(rows_v.at[b], acc.at[dstv_scat.at[b]],
                                  ssem.at[b]).wait()

        def copy_dstv(b):
            @pl.loop(0, CH // 16)
            def _(g):
                dstv_scat[b, pl.ds(g * 16, 16)] = dstv[b, pl.ds(g * 16, 16)]

        def compute_and_scatter(b):
            @pl.loop(0, CH // 16)
            def _(g):
                iota = lax.iota(jnp.int32, 16)
                e_vec = g * 16 + iota
                av = [plsc.load_gather(rows_v.at[b],
                                       [e_vec, jnp.full((16,), F + h, jnp.int32)])
                      for h in range(H)]
                bv = [plsc.load_gather(sdst16.at[b],
                                       [e_vec, jnp.full((16,), h, jnp.int32)])
                      for h in range(H)]
                zv = [a + bb for a, bb in zip(av, bv)]
                ev = [jnp.exp(jnp.maximum(z, z * ALPHA)) for z in zv]
                for h in range(H):
                    plsc.store_scatter(rows_v.at[b],
                                       [e_vec, jnp.full((16,), F + h, jnp.int32)],
                                       ev[h])

            @pl.loop(0, CH, unroll=4)
            def _(e):
                exvec = rows_v[b, e, pl.ds(F, 16)]
                ss = tuple(exvec[h] for h in range(H))
                for c in range(F // 16):
                    rows_v[b, e, pl.ds(c * 16, 16)] = (
                        rows_v[b, e, pl.ds(c * 16, 16)] * ss[c // 2])

            pltpu.async_copy(rows_v.at[b], acc.at[dstv_scat.at[b]],
                             ssem.at[b], add=True)

        def chunk_body(q, j_plus_2, guard_scatter):
            q1 = (q + 1) % 3
            wait_idx(q1)
            if guard_scatter is None:
                wait_scatter(q1)
            elif guard_scatter is not False:
                @pl.when(guard_scatter)
                def _():
                    wait_scatter(q1)
            issue_gathers(q1)
            wait_gathers(q)
            copy_dstv(q)
            if j_plus_2 is not None:
                issue_idx((q + 2) % 3, j_plus_2)
            compute_and_scatter(q)

        issue_idx(0, 0)
        issue_idx(1, 1)

        @pl.loop(0, CH)
        def _(r):
            for c in range(ROW // 16):
                rows_v[1, r, pl.ds(c * 16, 16)] = zero16

        wait_idx(0)
        issue_gathers(0)

        @pl.loop(0, nfull)
        def _(i):
            pltpu.async_copy(rows_v.at[1], acc.at[pl.ds(r0 + i * CH, CH)],
                             gsem_d.at[1])

        if rem:
            pltpu.async_copy(rows_v.at[1, pl.ds(0, rem)],
                             acc.at[pl.ds(r0 + nfull * CH, rem)], gsem_d.at[1])

        @pl.loop(0, nfull)
        def _(i):
            pltpu.make_async_copy(rows_v.at[1], acc.at[pl.ds(r0 + i * CH, CH)],
                                  gsem_d.at[1]).wait()

        if rem:
            pltpu.make_async_copy(rows_v.at[1, pl.ds(0, rem)],
                                  acc.at[pl.ds(r0 + nfull * CH, rem)],
                                  gsem_d.at[1]).wait()
        plsc.subcore_barrier()

        n_triples = n_chunks // 3

        @pl.loop(0, n_triples)
        def _(t):
            j0 = 3 * t
            chunk_body(0, j0 + 2, t > 0)
            chunk_body(1, j0 + 3, t > 0)
            chunk_body(2, j0 + 4, None)

        chunk_body(0, None, None)
        q = 1
        wait_scatter(2)
        wait_gathers(q)
        copy_dstv(q)
        compute_and_scatter(q)
        wait_scatter(0)
        wait_scatter(1)

        plsc.subcore_barrier()

        def drain_pair(rs, nr, issue):
            af = acc.at[pl.ds(rs, nr), pl.ds(0, F)]
            ad = acc.at[pl.ds(rs, nr), pl.ds(F, 16)]
            df = outf_hbm.at[cid, pl.ds(rs, nr)]
            dd = outd_hbm.at[cid, pl.ds(rs, nr)]
            if issue:
                pltpu.async_copy(af, df, gsem_r.at[0])
                pltpu.async_copy(ad, dd, gsem_d.at[0])
            else:
                pltpu.make_async_copy(af, df, gsem_r.at[0]).wait()
                pltpu.make_async_copy(ad, dd, gsem_d.at[0]).wait()

        for issue in (True, False):
            @pl.loop(0, nfull)
            def _(i):
                drain_pair(r0 + i * CH, CH, issue)

            if rem:
                drain_pair(r0 + nfull * CH, rem, issue)

    accf, accd = sc_pass(src, dst, s16, whaug)

    def comb_body(f_ref, d_ref, o_ref):
        s = f_ref[0] + f_ref[1]
        den = d_ref[0, :, :H] + d_ref[1, :, :H]
        r = 1.0 / (den + 1e-16)
        parts = [s[:, h * DH:(h + 1) * DH] * r[:, h:h + 1] for h in range(H)]
        o_ref[...] = jnp.concatenate(parts, axis=1)

    Bn = 1000
    hp = pl.pallas_call(
        comb_body,
        grid=(N // Bn,),
        in_specs=[pl.BlockSpec((NC, Bn, F), lambda i: (0, i, 0)),
                  pl.BlockSpec((NC, Bn, 16), lambda i: (0, i, 0))],
        out_specs=pl.BlockSpec((Bn, F), lambda i: (i, 0)),
        out_shape=jax.ShapeDtypeStruct((N, F), jnp.float32),
    )(accf, accd)
    return hp
